# baseline (speedup 1.0000x reference)
import functools

import jax
import jax.numpy as jnp
from jax import lax
from jax.experimental import pallas as pl
from jax.experimental.pallas import tpu as pltpu
from jax.experimental.pallas import tpu_sc as plsc

H = 512
W = 512
C = 128
N = 524288
HW = H * W

NC, NS, L = 2, 16, 16
NW = NC * NS
PTS_PER_W = N // NW
B = 128
NCHUNK = PTS_PER_W // B


def _transpose_body(x_ref, o_ref):
    o_ref[...] = x_ref[...].T


def _make_table(feat2d):
    TBLK = 2048
    return pl.pallas_call(
        _transpose_body,
        grid=(HW // TBLK,),
        in_specs=[pl.BlockSpec((C, TBLK), lambda i: (0, i))],
        out_specs=pl.BlockSpec((TBLK, C), lambda i: (i, 0)),
        out_shape=jax.ShapeDtypeStruct((HW, C), jnp.float32),
    )(feat2d)


def _sc_body(table, xs, ys, out, cbuf, ibuf, wbuf, rows, obuf, sem):
    wid = lax.axis_index("s") * NC + lax.axis_index("c")
    base = wid * PTS_PER_W

    def chunk(g, _):
        start = base + g * B
        pltpu.sync_copy(xs.at[pl.ds(start, B)], cbuf.at[0])
        pltpu.sync_copy(ys.at[pl.ds(start, B)], cbuf.at[1])

        for q in range(B // L):
            sl = pl.ds(q * L, L)
            x = cbuf[0, sl]
            y = cbuf[1, sl]
            fx = ((x + 1.0) * float(H) - 1.0) * 0.5
            fy = ((y + 1.0) * float(W) - 1.0) * 0.5
            tx = (fx + 1.0).astype(jnp.int32)
            ty = (fy + 1.0).astype(jnp.int32)
            x0 = tx - 1
            y0 = ty - 1
            wx1 = fx - x0.astype(jnp.float32)
            wy1 = fy - y0.astype(jnp.float32)
            wx0 = 1.0 - wx1
            wy0 = 1.0 - wy1
            wx0 = jnp.where(x0 >= 0, wx0, 0.0)
            wx1 = jnp.where(tx <= H - 1, wx1, 0.0)
            wy0 = jnp.where(y0 >= 0, wy0, 0.0)
            wy1 = jnp.where(ty <= W - 1, wy1, 0.0)
            x0c = jnp.maximum(x0, 0)
            x1c = jnp.minimum(tx, H - 1)
            y0c = jnp.maximum(y0, 0)
            y1c = jnp.minimum(ty, W - 1)
            b0 = x0c * W
            b1 = x1c * W
            ibuf[0, sl] = b0 + y0c
            ibuf[1, sl] = b0 + y1c
            ibuf[2, sl] = b1 + y0c
            ibuf[3, sl] = b1 + y1c
            wbuf[0, sl] = wx0 * wy0
            wbuf[1, sl] = wx0 * wy1
            wbuf[2, sl] = wx1 * wy0
            wbuf[3, sl] = wx1 * wy1

        cps = [pltpu.async_copy(table.at[ibuf.at[c]], rows.at[c], sem)
               for c in range(4)]
        for cp in cps:
            cp.wait()

        def combine(q, carry):
            go = pl.multiple_of(q * L, L)
            wv0 = wbuf[0, pl.ds(go, L)]
            wv1 = wbuf[1, pl.ds(go, L)]
            wv2 = wbuf[2, pl.ds(go, L)]
            wv3 = wbuf[3, pl.ds(go, L)]
            for pp in range(L):
                p = go + pp
                w00 = wv0[pp]
                w01 = wv1[pp]
                w10 = wv2[pp]
                w11 = wv3[pp]
                for j in range(C // L):
                    s = pl.ds(j * L, L)
                    obuf[p, s] = (rows[0, p, s] * w00 + rows[1, p, s] * w01
                                  + rows[2, p, s] * w10 + rows[3, p, s] * w11)
            return carry

        lax.fori_loop(0, B // L, combine, 0)

        pltpu.sync_copy(obuf, out.at[pl.ds(start, B)])
        return 0

    lax.fori_loop(0, NCHUNK, chunk, 0)


@jax.jit
def _sc_sample(table, xs, ys):
    mesh = plsc.VectorSubcoreMesh(core_axis_name="c", subcore_axis_name="s",
                                  num_cores=NC, num_subcores=NS)
    return pl.kernel(
        _sc_body,
        out_type=jax.ShapeDtypeStruct((N, C), jnp.float32),
        mesh=mesh,
        scratch_types=[
            pltpu.VMEM((2, B), jnp.float32),
            pltpu.VMEM((4, B), jnp.int32),
            pltpu.VMEM((4, B), jnp.float32),
            pltpu.VMEM((4, B, C), jnp.float32),
            pltpu.VMEM((B, C), jnp.float32),
            pltpu.SemaphoreType.DMA,
        ],
    )(table, xs, ys)


def kernel(coords, features):
    feat2d = features.reshape(C, HW)
    table = _make_table(feat2d)
    xs = coords[:, 0]
    ys = coords[:, 1]
    return _sc_sample(table, xs, ys)


# speedup vs baseline: 1.5032x; 1.5032x over previous
import jax
import jax.numpy as jnp
from jax import lax
from jax.experimental import pallas as pl
from jax.experimental.pallas import tpu as pltpu
from jax.experimental.pallas import tpu_sc as plsc

H = 512
W = 512
C = 128
N = 524288
HW = H * W

NC, NS, L = 2, 16, 16
NW = NC * NS
PTS_PER_W = N // NW
B = 64
NCHUNK = PTS_PER_W // B


def _transpose_body(x_ref, o_ref):
    o_ref[...] = x_ref[...].T


def _make_table(feat2d):
    TBLK = 2048
    return pl.pallas_call(
        _transpose_body,
        grid=(HW // TBLK,),
        in_specs=[pl.BlockSpec((C, TBLK), lambda i: (0, i))],
        out_specs=pl.BlockSpec((TBLK, C), lambda i: (i, 0)),
        out_shape=jax.ShapeDtypeStruct((HW, C), jnp.float32),
    )(feat2d)


def _sc_body(table, xs, ys, out, cbuf, ibuf, wbuf, rows, obuf,
             gsem0, gsem1, osem0, osem1, csem):
    wid = lax.axis_index("s") * NC + lax.axis_index("c")
    base = wid * PTS_PER_W
    gsems = (gsem0, gsem1)
    osems = (osem0, osem1)

    def fire_coords(g, slot):
        start = base + g * B
        pltpu.async_copy(xs.at[pl.ds(start, B)], cbuf.at[slot, 0], csem)
        pltpu.async_copy(ys.at[pl.ds(start, B)], cbuf.at[slot, 1], csem)

    def wait_coords(slot):
        pltpu.make_async_copy(xs.at[pl.ds(base, B)], cbuf.at[slot, 0],
                              csem).wait()
        pltpu.make_async_copy(ys.at[pl.ds(base, B)], cbuf.at[slot, 1],
                              csem).wait()

    def compute_iw(slot):
        for q in range(B // L):
            sl = pl.ds(q * L, L)
            x = cbuf[slot, 0, sl]
            y = cbuf[slot, 1, sl]
            fx = ((x + 1.0) * float(H) - 1.0) * 0.5
            fy = ((y + 1.0) * float(W) - 1.0) * 0.5
            tx = (fx + 1.0).astype(jnp.int32)
            ty = (fy + 1.0).astype(jnp.int32)
            x0 = tx - 1
            y0 = ty - 1
            wx1 = fx - x0.astype(jnp.float32)
            wy1 = fy - y0.astype(jnp.float32)
            wx0 = 1.0 - wx1
            wy0 = 1.0 - wy1
            wx0 = jnp.where(x0 >= 0, wx0, 0.0)
            wx1 = jnp.where(tx <= H - 1, wx1, 0.0)
            wy0 = jnp.where(y0 >= 0, wy0, 0.0)
            wy1 = jnp.where(ty <= W - 1, wy1, 0.0)
            x0c = jnp.maximum(x0, 0)
            x1c = jnp.minimum(tx, H - 1)
            y0c = jnp.maximum(y0, 0)
            y1c = jnp.minimum(ty, W - 1)
            b0 = x0c * W
            b1 = x1c * W
            ibuf[slot, 0, sl] = b0 + y0c
            ibuf[slot, 1, sl] = b0 + y1c
            ibuf[slot, 2, sl] = b1 + y0c
            ibuf[slot, 3, sl] = b1 + y1c
            wbuf[slot, 0, sl] = wx0 * wy0
            wbuf[slot, 1, sl] = wx0 * wy1
            wbuf[slot, 2, sl] = wx1 * wy0
            wbuf[slot, 3, sl] = wx1 * wy1

    def fire_gathers(slot):
        for c in range(4):
            pltpu.async_copy(table.at[ibuf.at[slot, c]],
                             rows.at[slot, c], gsems[slot])

    def wait_gathers(slot):
        for c in range(4):
            pltpu.make_async_copy(table.at[ibuf.at[slot, c]],
                                  rows.at[slot, c], gsems[slot]).wait()

    def combine(slot):
        def group(q, carry):
            go = pl.multiple_of(q * L, L)
            wv0 = wbuf[slot, 0, pl.ds(go, L)]
            wv1 = wbuf[slot, 1, pl.ds(go, L)]
            wv2 = wbuf[slot, 2, pl.ds(go, L)]
            wv3 = wbuf[slot, 3, pl.ds(go, L)]
            for pp in range(L):
                p = go + pp
                w00 = wv0[pp]
                w01 = wv1[pp]
                w10 = wv2[pp]
                w11 = wv3[pp]
                for j in range(C // L):
                    s = pl.ds(j * L, L)
                    obuf[slot, p, s] = (
                        rows[slot, 0, p, s] * w00 + rows[slot, 1, p, s] * w01
                        + rows[slot, 2, p, s] * w10 + rows[slot, 3, p, s] * w11)
            return carry

        lax.fori_loop(0, B // L, group, 0)

    def fire_out(g, slot):
        start = base + g * B
        pltpu.async_copy(obuf.at[slot], out.at[pl.ds(start, B)], osems[slot])

    def wait_out(slot):
        pltpu.make_async_copy(obuf.at[slot], out.at[pl.ds(base, B)],
                              osems[slot]).wait()

    fire_coords(0, 0)
    wait_coords(0)
    compute_iw(0)
    fire_gathers(0)
    fire_coords(1, 1)

    def outer(go, carry):
        for sub in range(2):
            g = go * 2 + sub
            slot = sub
            nslot = 1 - sub

            @pl.when(g + 1 < NCHUNK)
            def _prep():
                wait_coords(nslot)
                compute_iw(nslot)
                fire_gathers(nslot)

            @pl.when(g + 2 < NCHUNK)
            def _pref():
                fire_coords(g + 2, slot)

            wait_gathers(slot)

            @pl.when(g >= 2)
            def _drain():
                wait_out(slot)

            combine(slot)
            fire_out(g, slot)
        return carry

    lax.fori_loop(0, NCHUNK // 2, outer, 0)

    wait_out(0)
    wait_out(1)


@jax.jit
def _sc_sample(table, xs, ys):
    mesh = plsc.VectorSubcoreMesh(core_axis_name="c", subcore_axis_name="s",
                                  num_cores=NC, num_subcores=NS)
    return pl.kernel(
        _sc_body,
        out_type=jax.ShapeDtypeStruct((N, C), jnp.float32),
        mesh=mesh,
        scratch_types=[
            pltpu.VMEM((2, 2, B), jnp.float32),
            pltpu.VMEM((2, 4, B), jnp.int32),
            pltpu.VMEM((2, 4, B), jnp.float32),
            pltpu.VMEM((2, 4, B, C), jnp.float32),
            pltpu.VMEM((2, B, C), jnp.float32),
            pltpu.SemaphoreType.DMA,
            pltpu.SemaphoreType.DMA,
            pltpu.SemaphoreType.DMA,
            pltpu.SemaphoreType.DMA,
            pltpu.SemaphoreType.DMA,
        ],
    )(table, xs, ys)


def kernel(coords, features):
    feat2d = features.reshape(C, HW)
    table = _make_table(feat2d)
    return _sc_sample(table, coords[:, 0], coords[:, 1])
